# baseline (speedup 1.0000x reference)
import functools

import jax
import jax.numpy as jnp
from jax import lax
from jax.experimental import pallas as pl
from jax.experimental.pallas import tpu as pltpu
from jax.experimental.pallas import tpu_sc as plsc

_NC = 2
_NS = 16
_NW = _NC * _NS
_K = 128
_B = 16
_D = 128


def _sc_mesh():
    return plsc.VectorSubcoreMesh(
        core_axis_name="c", subcore_axis_name="s",
        num_cores=_NC, num_subcores=_NS)


def _sc_degree(row3, n_pad, nchunks):
    nt = n_pad // _NS

    @functools.partial(
        pl.kernel,
        out_type=jax.ShapeDtypeStruct((_NC, n_pad), jnp.float32),
        mesh=_sc_mesh(),
        scratch_types=[
            pltpu.VMEM((nchunks, _K), jnp.int32),
            pltpu.VMEM((_K,), jnp.float32),
            pltpu.VMEM((nt,), jnp.float32),
            pltpu.VMEM_SHARED((n_pad,), jnp.float32),
            pltpu.SemaphoreType.DMA,
        ],
    )
    def deg_kernel(row_hbm, cnt_hbm, idx_v, ones_v, z_v, cnt_sp, sem):
        c = lax.axis_index("c")
        s = lax.axis_index("s")
        wid = c * _NS + s

        def fill_ones(i, _):
            ones_v[pl.ds(i * 16, 16)] = jnp.full((16,), 1.0, jnp.float32)
            return 0
        lax.fori_loop(0, _K // 16, fill_ones, 0)

        def fill_zero(i, _):
            z_v[pl.ds(i * 16, 16)] = jnp.zeros((16,), jnp.float32)
            return 0
        lax.fori_loop(0, nt // 16, fill_zero, 0)

        pltpu.sync_copy(row_hbm.at[wid], idx_v)
        pltpu.sync_copy(z_v, cnt_sp.at[pl.ds(s * nt, nt)])
        plsc.subcore_barrier()

        def fire(t, _):
            pltpu.async_copy(ones_v, cnt_sp.at[idx_v.at[t]], sem, add=True)
            return 0
        lax.fori_loop(0, nchunks, fire, 0)

        def drain(t, _):
            pltpu.make_async_copy(ones_v, cnt_sp.at[idx_v.at[t]], sem).wait()
            return 0
        lax.fori_loop(0, nchunks, drain, 0)

        plsc.subcore_barrier()
        pltpu.sync_copy(cnt_sp.at[pl.ds(s * nt, nt)],
                        cnt_hbm.at[c, pl.ds(s * nt, nt)])

    return deg_kernel(row3)


def _sc_aggregate(y, row3, col3, n_pad, nchunks):
    nt = n_pad // _NS
    nblk = nchunks // _B

    @functools.partial(
        pl.kernel,
        out_type=jax.ShapeDtypeStruct((_NC, n_pad, _D), jnp.float32),
        mesh=_sc_mesh(),
        scratch_types=[
            pltpu.VMEM((_B, _K), jnp.int32),
            pltpu.VMEM((_B, _K), jnp.int32),
            pltpu.VMEM((_K, _D), jnp.float32),
            pltpu.VMEM((_K, _D), jnp.float32),
            pltpu.VMEM_SHARED((n_pad, _D), jnp.float32),
            pltpu.SemaphoreType.DMA,
            pltpu.SemaphoreType.DMA,
        ],
    )
    def agg_kernel(y_hbm, row_hbm, col_hbm, out_hbm,
                   cidx, ridx, rows0, rows1, acc_sp, sem0, sem1):
        c = lax.axis_index("c")
        s = lax.axis_index("s")
        wid = c * _NS + s

        def zrow(r, _):
            def zcol(l, _):
                rows0[r, pl.ds(l * 16, 16)] = jnp.zeros((16,), jnp.float32)
                return 0
            lax.fori_loop(0, _D // 16, zcol, 0)
            return 0
        lax.fori_loop(0, _K, zrow, 0)

        def zacc(b, _):
            pltpu.sync_copy(rows0, acc_sp.at[pl.ds(s * nt + b * _K, _K)])
            return 0
        lax.fori_loop(0, nt // _K, zacc, 0)

        plsc.subcore_barrier()

        def block(b, _):
            pltpu.sync_copy(col_hbm.at[wid, pl.ds(b * _B, _B)], cidx)
            pltpu.sync_copy(row_hbm.at[wid, pl.ds(b * _B, _B)], ridx)
            pltpu.async_copy(y_hbm.at[cidx.at[0]], rows0, sem0)

            def body(i, _):
                t0 = 2 * i
                pltpu.async_copy(y_hbm.at[cidx.at[t0 + 1]], rows1, sem1)
                pltpu.make_async_copy(y_hbm.at[cidx.at[t0]], rows0, sem0).wait()
                pltpu.sync_copy(rows0, acc_sp.at[ridx.at[t0]], add=True)

                @pl.when(i < _B // 2 - 1)
                def _():
                    pltpu.async_copy(y_hbm.at[cidx.at[t0 + 2]], rows0, sem0)

                pltpu.make_async_copy(y_hbm.at[cidx.at[t0 + 1]], rows1, sem1).wait()
                pltpu.sync_copy(rows1, acc_sp.at[ridx.at[t0 + 1]], add=True)
                return 0
            lax.fori_loop(0, _B // 2, body, 0)
            return 0
        lax.fori_loop(0, nblk, block, 0)

        plsc.subcore_barrier()
        pltpu.sync_copy(acc_sp.at[pl.ds(s * nt, nt)],
                        out_hbm.at[c, pl.ds(s * nt, nt)])

    return agg_kernel(y, row3, col3)


def _tc_transform(x_pad, W, cnt3):
    n_pad = x_pad.shape[0]
    blk = 256

    def body(x_ref, w_ref, cnt_ref, y_ref):
        xw = jnp.dot(x_ref[...], w_ref[...], preferred_element_type=jnp.float32)
        cnt = cnt_ref[...]
        dinv = lax.rsqrt(cnt[0] + cnt[1] + 2.0)
        y_ref[...] = xw * dinv

    return pl.pallas_call(
        body,
        grid=(n_pad // blk,),
        in_specs=[
            pl.BlockSpec((blk, _D), lambda i: (i, 0)),
            pl.BlockSpec((_D, _D), lambda i: (0, 0)),
            pl.BlockSpec((_NC, blk, 1), lambda i: (0, i, 0)),
        ],
        out_specs=pl.BlockSpec((blk, _D), lambda i: (i, 0)),
        out_shape=jax.ShapeDtypeStruct((n_pad, _D), jnp.float32),
    )(x_pad, W, cnt3)


def _tc_finalize(S, y, cnt3):
    n_pad = y.shape[0]
    blk = 256

    def body(s_ref, y_ref, cnt_ref, o_ref):
        cnt = cnt_ref[...]
        dinv = lax.rsqrt(cnt[0] + cnt[1] + 2.0)
        acc = s_ref[0] + s_ref[1] + 2.0 * y_ref[...]
        o_ref[...] = jnp.maximum(acc * dinv, 0.0)

    return pl.pallas_call(
        body,
        grid=(n_pad // blk,),
        in_specs=[
            pl.BlockSpec((_NC, blk, _D), lambda i: (0, i, 0)),
            pl.BlockSpec((blk, _D), lambda i: (i, 0)),
            pl.BlockSpec((_NC, blk, 1), lambda i: (0, i, 0)),
        ],
        out_specs=pl.BlockSpec((blk, _D), lambda i: (i, 0)),
        out_shape=jax.ShapeDtypeStruct((n_pad, _D), jnp.float32),
    )(S, y, cnt3)


def kernel(x, edge_index, W):
    n, d_in = x.shape
    e = edge_index.shape[1]

    n_pad = -(-n // (_NS * _K)) * (_NS * _K)
    e_pad = -(-e // (_NW * _K * _B)) * (_NW * _K * _B)
    nchunks = e_pad // (_NW * _K)

    row = edge_index[0]
    col = edge_index[1]
    pad_i = jnp.full((e_pad - e,), n_pad - 1, dtype=jnp.int32)
    row3 = jnp.concatenate([row, pad_i]).reshape(_NW, nchunks, _K)
    col3 = jnp.concatenate([col, pad_i]).reshape(_NW, nchunks, _K)
    x_p = jnp.pad(x, ((0, n_pad - n), (0, 0)))

    cnt = _sc_degree(row3, n_pad, nchunks)
    cnt3 = cnt[:, :, None]
    y = _tc_transform(x_p, W, cnt3)
    S = _sc_aggregate(y, row3, col3, n_pad, nchunks)
    out = _tc_finalize(S, y, cnt3)
    return out[:n]


# speedup vs baseline: 1.8829x; 1.8829x over previous
import functools

import jax
import jax.numpy as jnp
from jax import lax
from jax.experimental import pallas as pl
from jax.experimental.pallas import tpu as pltpu
from jax.experimental.pallas import tpu_sc as plsc

_NC = 2
_NS = 16
_NW = _NC * _NS
_K = 128
_B = 16
_D = 128
_H = _D // 2


def _sc_mesh():
    return plsc.VectorSubcoreMesh(
        core_axis_name="c", subcore_axis_name="s",
        num_cores=_NC, num_subcores=_NS)


def _sc_degree(row3, n_pad, nchunks):
    nt = n_pad // _NS

    @functools.partial(
        pl.kernel,
        out_type=jax.ShapeDtypeStruct((_NC, n_pad), jnp.float32),
        mesh=_sc_mesh(),
        scratch_types=[
            pltpu.VMEM((nchunks, _K), jnp.int32),
            pltpu.VMEM((_K,), jnp.float32),
            pltpu.VMEM((nt,), jnp.float32),
            pltpu.VMEM_SHARED((n_pad,), jnp.float32),
            pltpu.SemaphoreType.DMA,
        ],
    )
    def deg_kernel(row_hbm, cnt_hbm, idx_v, ones_v, z_v, cnt_sp, sem):
        c = lax.axis_index("c")
        s = lax.axis_index("s")
        wid = c * _NS + s

        def fill_ones(i, _):
            ones_v[pl.ds(i * 16, 16)] = jnp.full((16,), 1.0, jnp.float32)
            return 0
        lax.fori_loop(0, _K // 16, fill_ones, 0)

        def fill_zero(i, _):
            z_v[pl.ds(i * 16, 16)] = jnp.zeros((16,), jnp.float32)
            return 0
        lax.fori_loop(0, nt // 16, fill_zero, 0)

        pltpu.sync_copy(row_hbm.at[wid], idx_v)
        pltpu.sync_copy(z_v, cnt_sp.at[pl.ds(s * nt, nt)])
        plsc.subcore_barrier()

        def fire(t, _):
            pltpu.async_copy(ones_v, cnt_sp.at[idx_v.at[t]], sem, add=True)
            return 0
        lax.fori_loop(0, nchunks, fire, 0)

        def drain(t, _):
            pltpu.make_async_copy(ones_v, cnt_sp.at[idx_v.at[t]], sem).wait()
            return 0
        lax.fori_loop(0, nchunks, drain, 0)

        plsc.subcore_barrier()
        pltpu.sync_copy(cnt_sp.at[pl.ds(s * nt, nt)],
                        cnt_hbm.at[c, pl.ds(s * nt, nt)])

    return deg_kernel(row3)


def _sc_aggregate(y_lo, y_hi, row3, col3, n_pad, nchunks):
    nt = n_pad // _NS
    nblk = nchunks // _B

    @functools.partial(
        pl.kernel,
        out_type=jax.ShapeDtypeStruct((_NC, 2, n_pad, _H), jnp.float32),
        mesh=_sc_mesh(),
        scratch_types=[
            pltpu.VMEM((_B, _K), jnp.int32),
            pltpu.VMEM((_B, _K), jnp.int32),
            pltpu.VMEM((_K, _H), jnp.float32),
            pltpu.VMEM((_K, _H), jnp.float32),
            pltpu.VMEM_SHARED((n_pad, _H), jnp.float32),
            pltpu.VMEM_SHARED((n_pad, _H), jnp.float32),
            pltpu.SemaphoreType.DMA,
            pltpu.SemaphoreType.DMA,
        ],
        compiler_params=pltpu.CompilerParams(use_tc_tiling_on_sc=False),
    )
    def agg_kernel(ylo_hbm, yhi_hbm, row_hbm, col_hbm, out_hbm,
                   cidx, ridx, rows0, rows1, y_sp, acc_sp, sem0, sem1):
        c = lax.axis_index("c")
        s = lax.axis_index("s")
        wid = c * _NS + s

        def zrow(r, _):
            def zcol(l, _):
                rows0[r, pl.ds(l * 16, 16)] = jnp.zeros((16,), jnp.float32)
                return 0
            lax.fori_loop(0, _H // 16, zcol, 0)
            return 0
        lax.fori_loop(0, _K, zrow, 0)

        def one_pass(y_half_hbm, h):
            def stage(b, _):
                pltpu.sync_copy(y_half_hbm.at[pl.ds(s * nt + b * _K, _K)], rows1)
                pltpu.sync_copy(rows1, y_sp.at[pl.ds(s * nt + b * _K, _K)])
                return 0
            lax.fori_loop(0, nt // _K, stage, 0)
            def zacc(b, _):
                pltpu.sync_copy(rows0, acc_sp.at[pl.ds(s * nt + b * _K, _K)])
                return 0
            lax.fori_loop(0, nt // _K, zacc, 0)
            plsc.subcore_barrier()

            def block(b, _):
                pltpu.sync_copy(col_hbm.at[wid, pl.ds(b * _B, _B)], cidx)
                pltpu.sync_copy(row_hbm.at[wid, pl.ds(b * _B, _B)], ridx)
                pltpu.async_copy(y_sp.at[cidx.at[0]], rows0, sem0)

                def body(i, _):
                    t0 = 2 * i
                    pltpu.async_copy(y_sp.at[cidx.at[t0 + 1]], rows1, sem1)
                    pltpu.make_async_copy(y_sp.at[cidx.at[t0]], rows0, sem0).wait()
                    pltpu.sync_copy(rows0, acc_sp.at[ridx.at[t0]], add=True)

                    @pl.when(i < _B // 2 - 1)
                    def _():
                        pltpu.async_copy(y_sp.at[cidx.at[t0 + 2]], rows0, sem0)

                    pltpu.make_async_copy(y_sp.at[cidx.at[t0 + 1]], rows1, sem1).wait()
                    pltpu.sync_copy(rows1, acc_sp.at[ridx.at[t0 + 1]], add=True)
                    return 0
                lax.fori_loop(0, _B // 2, body, 0)
                return 0
            lax.fori_loop(0, nblk, block, 0)

            plsc.subcore_barrier()
            pltpu.sync_copy(acc_sp.at[pl.ds(s * nt, nt)],
                            out_hbm.at[c, h, pl.ds(s * nt, nt)])
            def rezero(r, _):
                def zc(l, _):
                    rows0[r, pl.ds(l * 16, 16)] = jnp.zeros((16,), jnp.float32)
                    return 0
                lax.fori_loop(0, _H // 16, zc, 0)
                return 0
            lax.fori_loop(0, _K, rezero, 0)

        one_pass(ylo_hbm, 0)
        one_pass(yhi_hbm, 1)

    return agg_kernel(y_lo, y_hi, row3, col3)


def _tc_transform(x_pad, W, cnt3):
    n_pad = x_pad.shape[0]
    blk = 256

    def body(x_ref, w_ref, cnt_ref, ylo_ref, yhi_ref):
        xw = jnp.dot(x_ref[...], w_ref[...], preferred_element_type=jnp.float32)
        cnt = cnt_ref[...]
        dinv = lax.rsqrt(cnt[0] + cnt[1] + 2.0)
        y = xw * dinv
        ylo_ref[...] = y[:, :_H]
        yhi_ref[...] = y[:, _H:]

    return pl.pallas_call(
        body,
        grid=(n_pad // blk,),
        in_specs=[
            pl.BlockSpec((blk, _D), lambda i: (i, 0)),
            pl.BlockSpec((_D, _D), lambda i: (0, 0)),
            pl.BlockSpec((_NC, blk, 1), lambda i: (0, i, 0)),
        ],
        out_specs=[
            pl.BlockSpec((blk, _H), lambda i: (i, 0)),
            pl.BlockSpec((blk, _H), lambda i: (i, 0)),
        ],
        out_shape=[
            jax.ShapeDtypeStruct((n_pad, _H), jnp.float32),
            jax.ShapeDtypeStruct((n_pad, _H), jnp.float32),
        ],
    )(x_pad, W, cnt3)


def _tc_finalize(S, y_lo, y_hi, cnt3):
    n_pad = y_lo.shape[0]
    blk = 256

    def body(s_ref, ylo_ref, yhi_ref, cnt_ref, o_ref):
        cnt = cnt_ref[...]
        dinv = lax.rsqrt(cnt[0] + cnt[1] + 2.0)
        acc_lo = s_ref[0, 0] + s_ref[1, 0] + 2.0 * ylo_ref[...]
        acc_hi = s_ref[0, 1] + s_ref[1, 1] + 2.0 * yhi_ref[...]
        o_ref[:, :_H] = jnp.maximum(acc_lo * dinv, 0.0)
        o_ref[:, _H:] = jnp.maximum(acc_hi * dinv, 0.0)

    return pl.pallas_call(
        body,
        grid=(n_pad // blk,),
        in_specs=[
            pl.BlockSpec((_NC, 2, blk, _H), lambda i: (0, 0, i, 0)),
            pl.BlockSpec((blk, _H), lambda i: (i, 0)),
            pl.BlockSpec((blk, _H), lambda i: (i, 0)),
            pl.BlockSpec((_NC, blk, 1), lambda i: (0, i, 0)),
        ],
        out_specs=pl.BlockSpec((blk, _D), lambda i: (i, 0)),
        out_shape=jax.ShapeDtypeStruct((n_pad, _D), jnp.float32),
    )(S, y_lo, y_hi, cnt3)


def kernel(x, edge_index, W):
    n, d_in = x.shape
    e = edge_index.shape[1]

    n_pad = -(-n // (_NS * _K)) * (_NS * _K)
    e_pad = -(-e // (_NW * _K * _B)) * (_NW * _K * _B)
    nchunks = e_pad // (_NW * _K)

    row = edge_index[0]
    col = edge_index[1]
    pad_i = jnp.full((e_pad - e,), n_pad - 1, dtype=jnp.int32)
    row3 = jnp.concatenate([row, pad_i]).reshape(_NW, nchunks, _K)
    col3 = jnp.concatenate([col, pad_i]).reshape(_NW, nchunks, _K)
    x_p = jnp.pad(x, ((0, n_pad - n), (0, 0)))

    cnt = _sc_degree(row3, n_pad, nchunks)
    cnt3 = cnt[:, :, None]
    y_lo, y_hi = _tc_transform(x_p, W, cnt3)
    S = _sc_aggregate(y_lo, y_hi, row3, col3, n_pad, nchunks)
    out = _tc_finalize(S, y_lo, y_hi, cnt3)
    return out[:n]


# speedup vs baseline: 2.2492x; 1.1945x over previous
import functools

import jax
import jax.numpy as jnp
from jax import lax
from jax.experimental import pallas as pl
from jax.experimental.pallas import tpu as pltpu
from jax.experimental.pallas import tpu_sc as plsc

_NC = 2
_NS = 16
_NW = _NC * _NS
_K = 128
_B = 16
_D = 128
_H = _D // 2


def _sc_mesh():
    return plsc.VectorSubcoreMesh(
        core_axis_name="c", subcore_axis_name="s",
        num_cores=_NC, num_subcores=_NS)


def _sc_degree(row3, n_pad, nchunks):
    nt = n_pad // _NS

    @functools.partial(
        pl.kernel,
        out_type=jax.ShapeDtypeStruct((_NC, n_pad), jnp.float32),
        mesh=_sc_mesh(),
        scratch_types=[
            pltpu.VMEM((nchunks, _K), jnp.int32),
            pltpu.VMEM((_K,), jnp.float32),
            pltpu.VMEM((nt,), jnp.float32),
            pltpu.VMEM_SHARED((n_pad,), jnp.float32),
            pltpu.SemaphoreType.DMA,
        ],
    )
    def deg_kernel(row_hbm, cnt_hbm, idx_v, ones_v, z_v, cnt_sp, sem):
        c = lax.axis_index("c")
        s = lax.axis_index("s")
        wid = c * _NS + s

        def fill_ones(i, _):
            ones_v[pl.ds(i * 16, 16)] = jnp.full((16,), 1.0, jnp.float32)
            return 0
        lax.fori_loop(0, _K // 16, fill_ones, 0)

        def fill_zero(i, _):
            z_v[pl.ds(i * 16, 16)] = jnp.zeros((16,), jnp.float32)
            return 0
        lax.fori_loop(0, nt // 16, fill_zero, 0)

        pltpu.sync_copy(row_hbm.at[wid], idx_v)
        pltpu.sync_copy(z_v, cnt_sp.at[pl.ds(s * nt, nt)])
        plsc.subcore_barrier()

        def fire(t, _):
            pltpu.async_copy(ones_v, cnt_sp.at[idx_v.at[t]], sem, add=True)
            return 0
        lax.fori_loop(0, nchunks, fire, 0)

        def drain(t, _):
            pltpu.make_async_copy(ones_v, cnt_sp.at[idx_v.at[t]], sem).wait()
            return 0
        lax.fori_loop(0, nchunks, drain, 0)

        plsc.subcore_barrier()
        pltpu.sync_copy(cnt_sp.at[pl.ds(s * nt, nt)],
                        cnt_hbm.at[c, pl.ds(s * nt, nt)])

    return deg_kernel(row3)


def _sc_aggregate(y_lo, y_hi, row3, col3, n_pad, nchunks):
    nt = n_pad // _NS
    nblk = nchunks // _B

    @functools.partial(
        pl.kernel,
        out_type=jax.ShapeDtypeStruct((_NC, n_pad, _H), jnp.float32),
        mesh=_sc_mesh(),
        scratch_types=[
            pltpu.VMEM((_B, _K), jnp.int32),
            pltpu.VMEM((_B, _K), jnp.int32),
            [pltpu.VMEM((_K, _H), jnp.float32)] * 4,
            pltpu.VMEM_SHARED((n_pad, _H), jnp.float32),
            pltpu.VMEM_SHARED((n_pad, _H), jnp.float32),
            [pltpu.SemaphoreType.DMA] * 4,
            [pltpu.SemaphoreType.DMA] * 4,
        ],
        compiler_params=pltpu.CompilerParams(use_tc_tiling_on_sc=False),
    )
    def agg_kernel(ylo_hbm, yhi_hbm, row_hbm, col_hbm, out_hbm,
                   cidx, ridx, rows, y_sp, acc_sp, gsem, ssem):
        c = lax.axis_index("c")
        s = lax.axis_index("s")

        def zrow(r, _):
            def zcol(l, _):
                rows[0][r, pl.ds(l * 16, 16)] = jnp.zeros((16,), jnp.float32)
                return 0
            lax.fori_loop(0, _H // 16, zcol, 0)
            return 0
        lax.fori_loop(0, _K, zrow, 0)

        def zacc(b, _):
            pltpu.sync_copy(rows[0], acc_sp.at[pl.ds(s * nt + b * _K, _K)])
            return 0
        lax.fori_loop(0, nt // _K, zacc, 0)

        def stage_from(y_half_hbm):
            def stage(b, _):
                pltpu.sync_copy(y_half_hbm.at[pl.ds(s * nt + b * _K, _K)], rows[1])
                pltpu.sync_copy(rows[1], y_sp.at[pl.ds(s * nt + b * _K, _K)])
                return 0
            lax.fori_loop(0, nt // _K, stage, 0)

        @pl.when(c == 0)
        def _():
            stage_from(ylo_hbm)

        @pl.when(c == 1)
        def _():
            stage_from(yhi_hbm)

        plsc.subcore_barrier()

        def block(b, _):
            pltpu.sync_copy(col_hbm.at[s, pl.ds(b * _B, _B)], cidx)
            pltpu.sync_copy(row_hbm.at[s, pl.ds(b * _B, _B)], ridx)
            pltpu.async_copy(y_sp.at[cidx.at[0]], rows[0], gsem[0])
            pltpu.async_copy(y_sp.at[cidx.at[1]], rows[1], gsem[1])
            for t in range(_B):
                j = t % 4
                pltpu.make_async_copy(y_sp.at[cidx.at[t]], rows[j], gsem[j]).wait()
                pltpu.async_copy(rows[j], acc_sp.at[ridx.at[t]], ssem[j], add=True)
                if t + 2 < _B:
                    jn = (t + 2) % 4
                    if t >= 2:
                        pltpu.make_async_copy(
                            rows[jn], acc_sp.at[ridx.at[t - 2]], ssem[jn]).wait()
                    pltpu.async_copy(y_sp.at[cidx.at[t + 2]], rows[jn], gsem[jn])
            for t in range(_B - 4, _B):
                pltpu.make_async_copy(
                    rows[t % 4], acc_sp.at[ridx.at[t]], ssem[t % 4]).wait()
            return 0
        lax.fori_loop(0, nblk, block, 0)

        plsc.subcore_barrier()
        pltpu.sync_copy(acc_sp.at[pl.ds(s * nt, nt)],
                        out_hbm.at[c, pl.ds(s * nt, nt)])

    return agg_kernel(y_lo, y_hi, row3, col3)


def _tc_transform(x_pad, W, cnt3):
    n_pad = x_pad.shape[0]
    blk = 256

    def body(x_ref, w_ref, cnt_ref, ylo_ref, yhi_ref):
        xw = jnp.dot(x_ref[...], w_ref[...], preferred_element_type=jnp.float32)
        cnt = cnt_ref[...]
        dinv = lax.rsqrt(cnt[0] + cnt[1] + 2.0)
        y = xw * dinv
        ylo_ref[...] = y[:, :_H]
        yhi_ref[...] = y[:, _H:]

    return pl.pallas_call(
        body,
        grid=(n_pad // blk,),
        in_specs=[
            pl.BlockSpec((blk, _D), lambda i: (i, 0)),
            pl.BlockSpec((_D, _D), lambda i: (0, 0)),
            pl.BlockSpec((_NC, blk, 1), lambda i: (0, i, 0)),
        ],
        out_specs=[
            pl.BlockSpec((blk, _H), lambda i: (i, 0)),
            pl.BlockSpec((blk, _H), lambda i: (i, 0)),
        ],
        out_shape=[
            jax.ShapeDtypeStruct((n_pad, _H), jnp.float32),
            jax.ShapeDtypeStruct((n_pad, _H), jnp.float32),
        ],
    )(x_pad, W, cnt3)


def _tc_finalize(S, y_lo, y_hi, cnt3):
    n_pad = y_lo.shape[0]
    blk = 256

    def body(s_ref, ylo_ref, yhi_ref, cnt_ref, o_ref):
        cnt = cnt_ref[...]
        dinv = lax.rsqrt(cnt[0] + cnt[1] + 2.0)
        acc_lo = s_ref[0] + 2.0 * ylo_ref[...]
        acc_hi = s_ref[1] + 2.0 * yhi_ref[...]
        o_ref[:, :_H] = jnp.maximum(acc_lo * dinv, 0.0)
        o_ref[:, _H:] = jnp.maximum(acc_hi * dinv, 0.0)

    return pl.pallas_call(
        body,
        grid=(n_pad // blk,),
        in_specs=[
            pl.BlockSpec((_NC, blk, _H), lambda i: (0, i, 0)),
            pl.BlockSpec((blk, _H), lambda i: (i, 0)),
            pl.BlockSpec((blk, _H), lambda i: (i, 0)),
            pl.BlockSpec((_NC, blk, 1), lambda i: (0, i, 0)),
        ],
        out_specs=pl.BlockSpec((blk, _D), lambda i: (i, 0)),
        out_shape=jax.ShapeDtypeStruct((n_pad, _D), jnp.float32),
    )(S, y_lo, y_hi, cnt3)


def kernel(x, edge_index, W):
    n, d_in = x.shape
    e = edge_index.shape[1]

    n_pad = -(-n // (_NS * _K)) * (_NS * _K)
    e_pad = -(-e // (_NS * _K * _B)) * (_NS * _K * _B)
    nchunks_deg = e_pad // (_NW * _K)
    nchunks_agg = e_pad // (_NS * _K)

    row = edge_index[0]
    col = edge_index[1]
    pad_i = jnp.full((e_pad - e,), n_pad - 1, dtype=jnp.int32)
    row_p = jnp.concatenate([row, pad_i])
    col_p = jnp.concatenate([col, pad_i])
    row3d = row_p.reshape(_NW, nchunks_deg, _K)
    row3a = row_p.reshape(_NS, nchunks_agg, _K)
    col3a = col_p.reshape(_NS, nchunks_agg, _K)
    x_p = jnp.pad(x, ((0, n_pad - n), (0, 0)))

    cnt = _sc_degree(row3d, n_pad, nchunks_deg)
    cnt3 = cnt[:, :, None]
    y_lo, y_hi = _tc_transform(x_p, W, cnt3)
    S = _sc_aggregate(y_lo, y_hi, row3a, col3a, n_pad, nchunks_agg)
    out = _tc_finalize(S, y_lo, y_hi, cnt3)
    return out[:n]


# speedup vs baseline: 2.2870x; 1.0168x over previous
import functools

import jax
import jax.numpy as jnp
from jax import lax
from jax.experimental import pallas as pl
from jax.experimental.pallas import tpu as pltpu
from jax.experimental.pallas import tpu_sc as plsc

_NC = 2
_NS = 16
_NW = _NC * _NS
_K = 128
_B = 16
_D = 128
_H = _D // 2


def _sc_mesh():
    return plsc.VectorSubcoreMesh(
        core_axis_name="c", subcore_axis_name="s",
        num_cores=_NC, num_subcores=_NS)


def _sc_degree(row3, n_pad, nchunks):
    nt = n_pad // _NS

    @functools.partial(
        pl.kernel,
        out_type=jax.ShapeDtypeStruct((_NC, n_pad), jnp.float32),
        mesh=_sc_mesh(),
        scratch_types=[
            pltpu.VMEM((nchunks, _K), jnp.int32),
            pltpu.VMEM((_K,), jnp.float32),
            pltpu.VMEM((nt,), jnp.float32),
            pltpu.VMEM_SHARED((n_pad,), jnp.float32),
            pltpu.SemaphoreType.DMA,
        ],
    )
    def deg_kernel(row_hbm, cnt_hbm, idx_v, ones_v, z_v, cnt_sp, sem):
        c = lax.axis_index("c")
        s = lax.axis_index("s")
        wid = c * _NS + s

        def fill_ones(i, _):
            ones_v[pl.ds(i * 16, 16)] = jnp.full((16,), 1.0, jnp.float32)
            return 0
        lax.fori_loop(0, _K // 16, fill_ones, 0)

        def fill_zero(i, _):
            z_v[pl.ds(i * 16, 16)] = jnp.zeros((16,), jnp.float32)
            return 0
        lax.fori_loop(0, nt // 16, fill_zero, 0)

        pltpu.sync_copy(row_hbm.at[wid], idx_v)
        pltpu.sync_copy(z_v, cnt_sp.at[pl.ds(s * nt, nt)])
        plsc.subcore_barrier()

        def fire(t, _):
            pltpu.async_copy(ones_v, cnt_sp.at[idx_v.at[t]], sem, add=True)
            return 0
        lax.fori_loop(0, nchunks, fire, 0)

        def drain(t, _):
            pltpu.make_async_copy(ones_v, cnt_sp.at[idx_v.at[t]], sem).wait()
            return 0
        lax.fori_loop(0, nchunks, drain, 0)

        plsc.subcore_barrier()
        pltpu.sync_copy(cnt_sp.at[pl.ds(s * nt, nt)],
                        cnt_hbm.at[c, pl.ds(s * nt, nt)])

    return deg_kernel(row3)


def _sc_aggregate(y_lo, y_hi, row3, col3, n_pad, nchunks):
    nt = n_pad // _NS
    nblk = nchunks // _B

    @functools.partial(
        pl.kernel,
        out_type=jax.ShapeDtypeStruct((_NC, n_pad, _H), jnp.float32),
        mesh=_sc_mesh(),
        scratch_types=[
            [pltpu.VMEM((_B, _K), jnp.int32)] * 2,
            [pltpu.VMEM((_B, _K), jnp.int32)] * 2,
            [pltpu.VMEM((_K, _H), jnp.float32)] * 4,
            pltpu.VMEM_SHARED((n_pad, _H), jnp.float32),
            pltpu.VMEM_SHARED((n_pad, _H), jnp.float32),
            [pltpu.SemaphoreType.DMA] * 4,
            [pltpu.SemaphoreType.DMA] * 4,
            pltpu.SemaphoreType.DMA,
        ],
        compiler_params=pltpu.CompilerParams(use_tc_tiling_on_sc=False),
    )
    def agg_kernel(ylo_hbm, yhi_hbm, row_hbm, col_hbm, out_hbm,
                   cidx2, ridx2, rows, y_sp, acc_sp, gsem, ssem, psem):
        c = lax.axis_index("c")
        s = lax.axis_index("s")

        def zrow(r, _):
            def zcol(l, _):
                rows[0][r, pl.ds(l * 16, 16)] = jnp.zeros((16,), jnp.float32)
                return 0
            lax.fori_loop(0, _H // 16, zcol, 0)
            return 0
        lax.fori_loop(0, _K, zrow, 0)

        def zacc(b, _):
            pltpu.async_copy(rows[0], acc_sp.at[pl.ds(s * nt + b * _K, _K)], psem)
            return 0
        lax.fori_loop(0, nt // _K, zacc, 0)

        def zacc_drain(b, _):
            pltpu.make_async_copy(
                rows[0], acc_sp.at[pl.ds(s * nt + b * _K, _K)], psem).wait()
            return 0
        lax.fori_loop(0, nt // _K, zacc_drain, 0)

        def stage_from(y_half_hbm):
            def stage(b, _):
                pltpu.sync_copy(y_half_hbm.at[pl.ds(s * nt + b * _K, _K)], rows[1])
                pltpu.sync_copy(rows[1], y_sp.at[pl.ds(s * nt + b * _K, _K)])
                return 0
            lax.fori_loop(0, nt // _K, stage, 0)

        @pl.when(c == 0)
        def _():
            stage_from(ylo_hbm)

        @pl.when(c == 1)
        def _():
            stage_from(yhi_hbm)

        plsc.subcore_barrier()

        pltpu.sync_copy(col_hbm.at[s, pl.ds(0, _B)], cidx2[0])
        pltpu.sync_copy(row_hbm.at[s, pl.ds(0, _B)], ridx2[0])

        def run_block(b, cidx, ridx, cidx_n, ridx_n):
            @pl.when(b + 1 < nblk)
            def _():
                pltpu.async_copy(col_hbm.at[s, pl.ds((b + 1) * _B, _B)], cidx_n, psem)
                pltpu.async_copy(row_hbm.at[s, pl.ds((b + 1) * _B, _B)], ridx_n, psem)

            pltpu.async_copy(y_sp.at[cidx.at[0]], rows[0], gsem[0])
            pltpu.async_copy(y_sp.at[cidx.at[1]], rows[1], gsem[1])
            for t in range(_B):
                j = t % 4
                pltpu.make_async_copy(y_sp.at[cidx.at[t]], rows[j], gsem[j]).wait()
                pltpu.async_copy(rows[j], acc_sp.at[ridx.at[t]], ssem[j], add=True)
                if t + 2 < _B:
                    jn = (t + 2) % 4
                    if t >= 2:
                        pltpu.make_async_copy(
                            rows[jn], acc_sp.at[ridx.at[t - 2]], ssem[jn]).wait()
                    pltpu.async_copy(y_sp.at[cidx.at[t + 2]], rows[jn], gsem[jn])
            for t in range(_B - 4, _B):
                pltpu.make_async_copy(
                    rows[t % 4], acc_sp.at[ridx.at[t]], ssem[t % 4]).wait()

            @pl.when(b + 1 < nblk)
            def _():
                pltpu.make_async_copy(
                    col_hbm.at[s, pl.ds((b + 1) * _B, _B)], cidx_n, psem).wait()
                pltpu.make_async_copy(
                    row_hbm.at[s, pl.ds((b + 1) * _B, _B)], ridx_n, psem).wait()

        def block_pair(i, _):
            run_block(2 * i, cidx2[0], ridx2[0], cidx2[1], ridx2[1])
            run_block(2 * i + 1, cidx2[1], ridx2[1], cidx2[0], ridx2[0])
            return 0
        lax.fori_loop(0, nblk // 2, block_pair, 0)

        plsc.subcore_barrier()
        pltpu.sync_copy(acc_sp.at[pl.ds(s * nt, nt)],
                        out_hbm.at[c, pl.ds(s * nt, nt)])

    return agg_kernel(y_lo, y_hi, row3, col3)


def _tc_transform(x_pad, W, cnt3):
    n_pad = x_pad.shape[0]
    blk = 256

    def body(x_ref, w_ref, cnt_ref, ylo_ref, yhi_ref):
        xw = jnp.dot(x_ref[...], w_ref[...], preferred_element_type=jnp.float32)
        cnt = cnt_ref[...]
        dinv = lax.rsqrt(cnt[0] + cnt[1] + 2.0)
        y = xw * dinv
        ylo_ref[...] = y[:, :_H]
        yhi_ref[...] = y[:, _H:]

    return pl.pallas_call(
        body,
        grid=(n_pad // blk,),
        in_specs=[
            pl.BlockSpec((blk, _D), lambda i: (i, 0)),
            pl.BlockSpec((_D, _D), lambda i: (0, 0)),
            pl.BlockSpec((_NC, blk, 1), lambda i: (0, i, 0)),
        ],
        out_specs=[
            pl.BlockSpec((blk, _H), lambda i: (i, 0)),
            pl.BlockSpec((blk, _H), lambda i: (i, 0)),
        ],
        out_shape=[
            jax.ShapeDtypeStruct((n_pad, _H), jnp.float32),
            jax.ShapeDtypeStruct((n_pad, _H), jnp.float32),
        ],
    )(x_pad, W, cnt3)


def _tc_finalize(S, y_lo, y_hi, cnt3):
    n_pad = y_lo.shape[0]
    blk = 256

    def body(s_ref, ylo_ref, yhi_ref, cnt_ref, o_ref):
        cnt = cnt_ref[...]
        dinv = lax.rsqrt(cnt[0] + cnt[1] + 2.0)
        acc_lo = s_ref[0] + 2.0 * ylo_ref[...]
        acc_hi = s_ref[1] + 2.0 * yhi_ref[...]
        o_ref[:, :_H] = jnp.maximum(acc_lo * dinv, 0.0)
        o_ref[:, _H:] = jnp.maximum(acc_hi * dinv, 0.0)

    return pl.pallas_call(
        body,
        grid=(n_pad // blk,),
        in_specs=[
            pl.BlockSpec((_NC, blk, _H), lambda i: (0, i, 0)),
            pl.BlockSpec((blk, _H), lambda i: (i, 0)),
            pl.BlockSpec((blk, _H), lambda i: (i, 0)),
            pl.BlockSpec((_NC, blk, 1), lambda i: (0, i, 0)),
        ],
        out_specs=pl.BlockSpec((blk, _D), lambda i: (i, 0)),
        out_shape=jax.ShapeDtypeStruct((n_pad, _D), jnp.float32),
    )(S, y_lo, y_hi, cnt3)


def kernel(x, edge_index, W):
    n, d_in = x.shape
    e = edge_index.shape[1]

    n_pad = -(-n // (_NS * _K)) * (_NS * _K)
    e_pad = -(-e // (_NS * _K * _B)) * (_NS * _K * _B)
    nchunks_deg = e_pad // (_NW * _K)
    nchunks_agg = e_pad // (_NS * _K)

    row = edge_index[0]
    col = edge_index[1]
    pad_i = jnp.full((e_pad - e,), n_pad - 1, dtype=jnp.int32)
    row_p = jnp.concatenate([row, pad_i])
    col_p = jnp.concatenate([col, pad_i])
    row3d = row_p.reshape(_NW, nchunks_deg, _K)
    row3a = row_p.reshape(_NS, nchunks_agg, _K)
    col3a = col_p.reshape(_NS, nchunks_agg, _K)
    x_p = jnp.pad(x, ((0, n_pad - n), (0, 0)))

    cnt = _sc_degree(row3d, n_pad, nchunks_deg)
    cnt3 = cnt[:, :, None]
    y_lo, y_hi = _tc_transform(x_p, W, cnt3)
    S = _sc_aggregate(y_lo, y_hi, row3a, col3a, n_pad, nchunks_agg)
    out = _tc_finalize(S, y_lo, y_hi, cnt3)
    return out[:n]


# speedup vs baseline: 2.4735x; 1.0815x over previous
import functools

import jax
import jax.numpy as jnp
from jax import lax
from jax.experimental import pallas as pl
from jax.experimental.pallas import tpu as pltpu
from jax.experimental.pallas import tpu_sc as plsc

_NC = 2
_NS = 16
_NW = _NC * _NS
_K = 128
_B = 16
_D = 128
_H = _D // 2


def _sc_mesh():
    return plsc.VectorSubcoreMesh(
        core_axis_name="c", subcore_axis_name="s",
        num_cores=_NC, num_subcores=_NS)


def _sc_degree(row3, n_pad, nchunks):
    nt = n_pad // _NS

    @functools.partial(
        pl.kernel,
        out_type=jax.ShapeDtypeStruct((_NC, n_pad), jnp.float32),
        mesh=_sc_mesh(),
        scratch_types=[
            pltpu.VMEM((nchunks, _K), jnp.int32),
            pltpu.VMEM((_K,), jnp.float32),
            pltpu.VMEM((nt,), jnp.float32),
            pltpu.VMEM_SHARED((n_pad,), jnp.float32),
            pltpu.SemaphoreType.DMA,
        ],
    )
    def deg_kernel(row_hbm, cnt_hbm, idx_v, ones_v, z_v, cnt_sp, sem):
        c = lax.axis_index("c")
        s = lax.axis_index("s")
        wid = c * _NS + s

        def fill_ones(i, _):
            ones_v[pl.ds(i * 16, 16)] = jnp.full((16,), 1.0, jnp.float32)
            return 0
        lax.fori_loop(0, _K // 16, fill_ones, 0)

        def fill_zero(i, _):
            z_v[pl.ds(i * 16, 16)] = jnp.zeros((16,), jnp.float32)
            return 0
        lax.fori_loop(0, nt // 16, fill_zero, 0)

        pltpu.sync_copy(row_hbm.at[wid], idx_v)
        pltpu.sync_copy(z_v, cnt_sp.at[pl.ds(s * nt, nt)])
        plsc.subcore_barrier()

        def fire(t, _):
            pltpu.async_copy(ones_v, cnt_sp.at[idx_v.at[t]], sem, add=True)
            return 0
        lax.fori_loop(0, nchunks, fire, 0)

        def drain(t, _):
            pltpu.make_async_copy(ones_v, cnt_sp.at[idx_v.at[t]], sem).wait()
            return 0
        lax.fori_loop(0, nchunks, drain, 0)

        plsc.subcore_barrier()
        pltpu.sync_copy(cnt_sp.at[pl.ds(s * nt, nt)],
                        cnt_hbm.at[c, pl.ds(s * nt, nt)])

    return deg_kernel(row3)


def _sc_aggregate(y_lo, y_hi, row3, col3, n_pad, nchunks):
    nt = n_pad // _NS
    nblk = nchunks // _B

    @functools.partial(
        pl.kernel,
        out_type=jax.ShapeDtypeStruct((_NC, n_pad, _H), jnp.float32),
        mesh=_sc_mesh(),
        scratch_types=[
            [pltpu.VMEM((_B, _K), jnp.int32)] * 2,
            [pltpu.VMEM((_B, _K), jnp.int32)] * 2,
            [pltpu.VMEM((_K, _H), jnp.float32)] * 4,
            pltpu.VMEM_SHARED((n_pad, _H), jnp.float32),
            pltpu.VMEM_SHARED((n_pad, _H), jnp.float32),
            [pltpu.SemaphoreType.DMA] * 4,
            [pltpu.SemaphoreType.DMA] * 4,
            pltpu.SemaphoreType.DMA,
        ],
        compiler_params=pltpu.CompilerParams(use_tc_tiling_on_sc=False),
    )
    def agg_kernel(ylo_hbm, yhi_hbm, row_hbm, col_hbm, out_hbm,
                   cidx2, ridx2, rows, y_sp, acc_sp, gsem, ssem, psem):
        c = lax.axis_index("c")
        s = lax.axis_index("s")

        def zrow(r, _):
            def zcol(l, _):
                rows[0][r, pl.ds(l * 16, 16)] = jnp.zeros((16,), jnp.float32)
                return 0
            lax.fori_loop(0, _H // 16, zcol, 0)
            return 0
        lax.fori_loop(0, _K, zrow, 0)

        def zacc(b, _):
            pltpu.async_copy(rows[0], acc_sp.at[pl.ds(s * nt + b * _K, _K)], psem)
            return 0
        lax.fori_loop(0, nt // _K, zacc, 0)

        def zacc_drain(b, _):
            pltpu.make_async_copy(
                rows[0], acc_sp.at[pl.ds(s * nt + b * _K, _K)], psem).wait()
            return 0
        lax.fori_loop(0, nt // _K, zacc_drain, 0)

        def stage_from(y_half_hbm):
            def stage(b, _):
                pltpu.sync_copy(y_half_hbm.at[pl.ds(s * nt + b * _K, _K)], rows[1])
                pltpu.sync_copy(rows[1], y_sp.at[pl.ds(s * nt + b * _K, _K)])
                return 0
            lax.fori_loop(0, nt // _K, stage, 0)

        @pl.when(c == 0)
        def _():
            stage_from(ylo_hbm)

        @pl.when(c == 1)
        def _():
            stage_from(yhi_hbm)

        plsc.subcore_barrier()

        pltpu.sync_copy(col_hbm.at[s, pl.ds(0, _B)], cidx2[0])
        pltpu.sync_copy(row_hbm.at[s, pl.ds(0, _B)], ridx2[0])

        def run_block(b, cidx, ridx, cidx_n, ridx_n):
            @pl.when(b + 1 < nblk)
            def _():
                pltpu.async_copy(col_hbm.at[s, pl.ds((b + 1) * _B, _B)], cidx_n, psem)
                pltpu.async_copy(row_hbm.at[s, pl.ds((b + 1) * _B, _B)], ridx_n, psem)

            pltpu.async_copy(y_sp.at[cidx.at[0]], rows[0], gsem[0])
            pltpu.async_copy(y_sp.at[cidx.at[1]], rows[1], gsem[1])
            for t in range(_B):
                j = t % 4
                pltpu.make_async_copy(y_sp.at[cidx.at[t]], rows[j], gsem[j]).wait()
                pltpu.async_copy(rows[j], acc_sp.at[ridx.at[t]], ssem[j], add=True)
                if t + 2 < _B:
                    jn = (t + 2) % 4
                    if t >= 2:
                        pltpu.make_async_copy(
                            rows[jn], acc_sp.at[ridx.at[t - 2]], ssem[jn]).wait()
                    pltpu.async_copy(y_sp.at[cidx.at[t + 2]], rows[jn], gsem[jn])
            for t in range(_B - 4, _B):
                pltpu.make_async_copy(
                    rows[t % 4], acc_sp.at[ridx.at[t]], ssem[t % 4]).wait()

            @pl.when(b + 1 < nblk)
            def _():
                pltpu.make_async_copy(
                    col_hbm.at[s, pl.ds((b + 1) * _B, _B)], cidx_n, psem).wait()
                pltpu.make_async_copy(
                    row_hbm.at[s, pl.ds((b + 1) * _B, _B)], ridx_n, psem).wait()

        def block_pair(i, _):
            run_block(2 * i, cidx2[0], ridx2[0], cidx2[1], ridx2[1])
            run_block(2 * i + 1, cidx2[1], ridx2[1], cidx2[0], ridx2[0])
            return 0
        lax.fori_loop(0, nblk // 2, block_pair, 0)

        plsc.subcore_barrier()
        pltpu.sync_copy(acc_sp.at[pl.ds(s * nt, nt)],
                        out_hbm.at[c, pl.ds(s * nt, nt)])

    return agg_kernel(y_lo, y_hi, row3, col3)


def _tc_transform(x_pad, W, cnt3):
    n_pad = x_pad.shape[0]
    blk = 512

    def body(x_ref, w_ref, cnt_ref, ylo_ref, yhi_ref):
        xw = jnp.dot(x_ref[...], w_ref[...], preferred_element_type=jnp.float32)
        cnt = cnt_ref[...]
        dinv = lax.rsqrt(cnt[0] + cnt[1] + 2.0)
        y = xw * dinv
        ylo_ref[...] = y[:, :_H]
        yhi_ref[...] = y[:, _H:]

    return pl.pallas_call(
        body,
        grid=(n_pad // blk,),
        in_specs=[
            pl.BlockSpec((blk, _D), lambda i: (i, 0)),
            pl.BlockSpec((_D, _D), lambda i: (0, 0)),
            pl.BlockSpec((_NC, blk, 1), lambda i: (0, i, 0)),
        ],
        out_specs=[
            pl.BlockSpec((blk, _H), lambda i: (i, 0)),
            pl.BlockSpec((blk, _H), lambda i: (i, 0)),
        ],
        out_shape=[
            jax.ShapeDtypeStruct((n_pad, _H), jnp.float32),
            jax.ShapeDtypeStruct((n_pad, _H), jnp.float32),
        ],
    )(x_pad, W, cnt3)


def _tc_finalize(S, y_lo, y_hi, cnt3):
    n_pad = y_lo.shape[0]
    blk = 512

    def body(s_ref, ylo_ref, yhi_ref, cnt_ref, o_ref):
        cnt = cnt_ref[...]
        dinv = lax.rsqrt(cnt[0] + cnt[1] + 2.0)
        acc_lo = s_ref[0] + 2.0 * ylo_ref[...]
        acc_hi = s_ref[1] + 2.0 * yhi_ref[...]
        o_ref[:, :_H] = jnp.maximum(acc_lo * dinv, 0.0)
        o_ref[:, _H:] = jnp.maximum(acc_hi * dinv, 0.0)

    return pl.pallas_call(
        body,
        grid=(n_pad // blk,),
        in_specs=[
            pl.BlockSpec((_NC, blk, _H), lambda i: (0, i, 0)),
            pl.BlockSpec((blk, _H), lambda i: (i, 0)),
            pl.BlockSpec((blk, _H), lambda i: (i, 0)),
            pl.BlockSpec((_NC, blk, 1), lambda i: (0, i, 0)),
        ],
        out_specs=pl.BlockSpec((blk, _D), lambda i: (i, 0)),
        out_shape=jax.ShapeDtypeStruct((n_pad, _D), jnp.float32),
    )(S, y_lo, y_hi, cnt3)


def kernel(x, edge_index, W):
    n, d_in = x.shape
    e = edge_index.shape[1]

    n_pad = -(-n // (_NS * _K)) * (_NS * _K)
    e_pad = -(-e // (_NS * _K * _B)) * (_NS * _K * _B)
    nchunks_deg = e_pad // (_NW * _K)
    nchunks_agg = e_pad // (_NS * _K)

    row = edge_index[0]
    col = edge_index[1]
    pad_i = jnp.full((e_pad - e,), n_pad - 1, dtype=jnp.int32)
    row_p = jnp.concatenate([row, pad_i])
    col_p = jnp.concatenate([col, pad_i])
    row3d = row_p.reshape(_NW, nchunks_deg, _K)
    row3a = row_p.reshape(_NS, nchunks_agg, _K)
    col3a = col_p.reshape(_NS, nchunks_agg, _K)
    x_p = jnp.pad(x, ((0, n_pad - n), (0, 0)))

    cnt = _sc_degree(row3d, n_pad, nchunks_deg)
    cnt3 = cnt[:, :, None]
    y_lo, y_hi = _tc_transform(x_p, W, cnt3)
    S = _sc_aggregate(y_lo, y_hi, row3a, col3a, n_pad, nchunks_agg)
    out = _tc_finalize(S, y_lo, y_hi, cnt3)
    return out[:n]
